# Initial kernel scaffold; baseline (speedup 1.0000x reference)
import functools

import jax
import jax.numpy as jnp
from jax import lax
from jax.experimental import pallas as pl
from jax.experimental.pallas import tpu as pltpu
from jax.experimental.pallas import tpu_sc as plsc

N = 100000
E = 1600000
F_IN = 4
H = 64
G = 1024

C = 2000
EPT32 = E // 32
EPT16 = E // 16
ROW_BIG = 6272
ROW_LAST = N - 15 * ROW_BIG

_mesh = plsc.VectorSubcoreMesh(core_axis_name="c", subcore_axis_name="s")


def _rows_copy(do_copy):
    s = lax.axis_index("s")

    @pl.when(s < 15)
    def _():
        do_copy(s * ROW_BIG, ROW_BIG)

    @pl.when(s == 15)
    def _():
        do_copy(15 * ROW_BIG, ROW_LAST)


@functools.partial(
    pl.kernel, mesh=_mesh,
    out_type=jax.ShapeDtypeStruct((2, N), jnp.float32),
    scratch_types=[
        pltpu.VMEM((C,), jnp.int32),
        pltpu.VMEM((C,), jnp.float32),
        pltpu.VMEM_SHARED((N,), jnp.float32),
    ],
)
def _deg_kernel(dst_hbm, zeros_hbm, ones_hbm, out_hbm, dst_v, ones_v, acc):
    c = lax.axis_index("c")
    s = lax.axis_index("s")
    wid = s * 2 + c
    pltpu.sync_copy(ones_hbm, ones_v)
    _rows_copy(lambda off, n: pltpu.sync_copy(
        zeros_hbm.at[pl.ds(off, n)], acc.at[pl.ds(off, n)]))
    plsc.subcore_barrier()

    def body(i, carry):
        off = wid * EPT32 + i * C
        pltpu.sync_copy(dst_hbm.at[pl.ds(off, C)], dst_v)
        pltpu.sync_copy(ones_v, acc.at[dst_v], add=True)
        return carry

    lax.fori_loop(0, EPT32 // C, body, 0)
    plsc.subcore_barrier()
    _rows_copy(lambda off, n: pltpu.sync_copy(
        acc.at[pl.ds(off, n)], out_hbm.at[c, pl.ds(off, n)]))


@functools.partial(
    pl.kernel, mesh=_mesh,
    out_type=jax.ShapeDtypeStruct((2, N, F_IN), jnp.float32),
    scratch_types=[
        pltpu.VMEM((C,), jnp.int32),
        pltpu.VMEM((C,), jnp.int32),
        pltpu.VMEM((C, F_IN), jnp.float32),
        pltpu.VMEM_SHARED((N, F_IN), jnp.float32),
        pltpu.SemaphoreType.DMA,
    ],
)
def _agg1_kernel(src_hbm, dst_hbm, xp_hbm, zeros_hbm, out_hbm,
                 src_v, dst_v, rows_v, acc, sem):
    c = lax.axis_index("c")
    s = lax.axis_index("s")
    wid = s * 2 + c
    _rows_copy(lambda off, n: pltpu.sync_copy(
        zeros_hbm.at[pl.ds(off, n), :], acc.at[pl.ds(off, n), :]))
    plsc.subcore_barrier()

    def body(i, carry):
        off = wid * EPT32 + i * C
        pltpu.sync_copy(src_hbm.at[pl.ds(off, C)], src_v)
        pltpu.sync_copy(dst_hbm.at[pl.ds(off, C)], dst_v)
        pltpu.async_copy(xp_hbm.at[src_v], rows_v, sem).wait()
        pltpu.sync_copy(rows_v, acc.at[dst_v], add=True)
        return carry

    lax.fori_loop(0, EPT32 // C, body, 0)
    plsc.subcore_barrier()
    _rows_copy(lambda off, n: pltpu.sync_copy(
        acc.at[pl.ds(off, n), :], out_hbm.at[c, pl.ds(off, n), :]))


@functools.partial(
    pl.kernel, mesh=_mesh,
    out_type=tuple(jax.ShapeDtypeStruct((N, 16), jnp.float32)
                   for _ in range(4)),
    scratch_types=[
        pltpu.VMEM((C,), jnp.int32),
        pltpu.VMEM((C,), jnp.int32),
        pltpu.VMEM((C, 16), jnp.float32),
        pltpu.VMEM_SHARED((N, 16), jnp.float32),
        pltpu.SemaphoreType.DMA,
    ],
)
def _agg2_kernel(src_hbm, dst_hbm, h0_hbm, h1_hbm, h2_hbm, h3_hbm,
                 t0_hbm, t1_hbm, t2_hbm, t3_hbm,
                 src_v, dst_v, rows_v, acc, sem):
    c = lax.axis_index("c")
    s = lax.axis_index("s")
    h_refs = (h0_hbm, h1_hbm, h2_hbm, h3_hbm)
    t_refs = (t0_hbm, t1_hbm, t2_hbm, t3_hbm)
    for chunk in range(4):
        @pl.when(c == chunk // 2)
        def _(chunk=chunk):
            h_hbm = h_refs[chunk]
            t_hbm = t_refs[chunk]
            _rows_copy(lambda off, n: pltpu.sync_copy(
                h_hbm.at[pl.ds(off, n), :], acc.at[pl.ds(off, n), :]))
            plsc.subcore_barrier()

            def body(i, carry):
                off = s * EPT16 + i * C
                pltpu.sync_copy(src_hbm.at[pl.ds(off, C)], src_v)
                pltpu.sync_copy(dst_hbm.at[pl.ds(off, C)], dst_v)
                pltpu.async_copy(h_hbm.at[src_v], rows_v, sem).wait()
                pltpu.sync_copy(rows_v, acc.at[dst_v], add=True)
                return carry

            lax.fori_loop(0, EPT16 // C, body, 0)
            plsc.subcore_barrier()
            _rows_copy(lambda off, n: pltpu.sync_copy(
                acc.at[pl.ds(off, n), :], t_hbm.at[pl.ds(off, n), :]))


_RB = 2000
_NBLK = N // _RB


def _prep_body(degp_ref, x_ref, dinv_ref, xp_ref):
    deg = degp_ref[0, :] + degp_ref[1, :] + 1.0
    di = lax.rsqrt(deg)
    dinv_ref[...] = di[:, None]
    xp_ref[...] = x_ref[...] * di[:, None]


def _mid_body(sp_ref, xp_ref, dinv_ref, w1_ref, b1_ref, *out_refs):
    di = dinv_ref[...]
    sfull = sp_ref[0] + sp_ref[1] + xp_ref[...]
    z = jnp.dot(di * sfull, w1_ref[...],
                preferred_element_type=jnp.float32) + b1_ref[...]
    h = jnp.where(z > 0, z, jnp.exp(jnp.minimum(z, 0.0)) - 1.0)
    h1p = di * h
    for k in range(4):
        out_refs[k][...] = h1p[:, 16 * k:16 * (k + 1)]


def _out_body(t0_ref, t1_ref, t2_ref, t3_ref, dinv_ref, batch_ref,
              w2_ref, b2_ref, w3_ref, b3_ref, out_ref, s_acc, c_acc):
    i = pl.program_id(0)
    t = jnp.concatenate(
        [t0_ref[...], t1_ref[...], t2_ref[...], t3_ref[...]], axis=1)
    z = jnp.dot(dinv_ref[...] * t, w2_ref[...],
                preferred_element_type=jnp.float32) + b2_ref[...]
    h2 = jnp.where(z > 0, z, jnp.exp(jnp.minimum(z, 0.0)) - 1.0)
    p = jnp.dot(h2, w3_ref[...], preferred_element_type=jnp.float32)
    onehot = (batch_ref[...] == lax.broadcasted_iota(
        jnp.int32, (_RB, G), 1)).astype(jnp.float32)
    sp = lax.dot_general(onehot, p, (((0,), (0,)), ((), ())),
                         preferred_element_type=jnp.float32)
    cp = jnp.sum(onehot, axis=0)[:, None]

    @pl.when(i == 0)
    def _():
        s_acc[...] = sp
        c_acc[...] = cp

    @pl.when(i > 0)
    def _():
        s_acc[...] += sp
        c_acc[...] += cp

    @pl.when(i == _NBLK - 1)
    def _():
        out_ref[...] = s_acc[...] / jnp.maximum(c_acc[...], 1.0) + b3_ref[...]


def kernel(x, edge_index, batch, W1, b1, W2, b2, W3, b3):
    src = edge_index[0].astype(jnp.int32)
    dst = edge_index[1].astype(jnp.int32)
    batch2 = batch.astype(jnp.int32)[:, None]
    z1 = jnp.zeros((N,), jnp.float32)
    z4 = jnp.zeros((N, F_IN), jnp.float32)
    ones_c = jnp.ones((C,), jnp.float32)
    b1r = b1[None, :]
    b2r = b2[None, :]
    b3r = b3[None, :]

    degp = _deg_kernel(dst, z1, ones_c)

    dinv, xp = pl.pallas_call(
        _prep_body,
        grid=(_NBLK,),
        in_specs=[
            pl.BlockSpec((2, _RB), lambda i: (0, i)),
            pl.BlockSpec((_RB, F_IN), lambda i: (i, 0)),
        ],
        out_specs=[
            pl.BlockSpec((_RB, 1), lambda i: (i, 0)),
            pl.BlockSpec((_RB, F_IN), lambda i: (i, 0)),
        ],
        out_shape=[
            jax.ShapeDtypeStruct((N, 1), jnp.float32),
            jax.ShapeDtypeStruct((N, F_IN), jnp.float32),
        ],
    )(degp, x)

    sp = _agg1_kernel(src, dst, xp, z4)

    h1p = pl.pallas_call(
        _mid_body,
        grid=(_NBLK,),
        in_specs=[
            pl.BlockSpec((2, _RB, F_IN), lambda i: (0, i, 0)),
            pl.BlockSpec((_RB, F_IN), lambda i: (i, 0)),
            pl.BlockSpec((_RB, 1), lambda i: (i, 0)),
            pl.BlockSpec((F_IN, H), lambda i: (0, 0)),
            pl.BlockSpec((1, H), lambda i: (0, 0)),
        ],
        out_specs=[pl.BlockSpec((_RB, 16), lambda i: (i, 0))] * 4,
        out_shape=[jax.ShapeDtypeStruct((N, 16), jnp.float32)] * 4,
    )(sp, xp, dinv, W1, b1r)

    t = _agg2_kernel(src, dst, *h1p)

    out = pl.pallas_call(
        _out_body,
        grid=(_NBLK,),
        in_specs=[pl.BlockSpec((_RB, 16), lambda i: (i, 0))] * 4 + [
            pl.BlockSpec((_RB, 1), lambda i: (i, 0)),
            pl.BlockSpec((_RB, 1), lambda i: (i, 0)),
            pl.BlockSpec((H, H), lambda i: (0, 0)),
            pl.BlockSpec((1, H), lambda i: (0, 0)),
            pl.BlockSpec((H, 1), lambda i: (0, 0)),
            pl.BlockSpec((1, 1), lambda i: (0, 0)),
        ],
        out_specs=pl.BlockSpec((G, 1), lambda i: (0, 0)),
        out_shape=jax.ShapeDtypeStruct((G, 1), jnp.float32),
        scratch_shapes=[
            pltpu.VMEM((G, 1), jnp.float32),
            pltpu.VMEM((G, 1), jnp.float32),
        ],
    )(*t, dinv, batch2, W2, b2r, W3, b3r)

    return out


# speedup vs baseline: 25.8593x; 25.8593x over previous
import functools

import jax
import jax.numpy as jnp
from jax import lax
from jax.experimental import pallas as pl
from jax.experimental.pallas import tpu as pltpu
from jax.experimental.pallas import tpu_sc as plsc

N = 100000
E = 1600000
F_IN = 4
H = 64
G = 1024

CE = 800
E_PAD = 1638400
TRASH = 1024
C = 2000
ROW_BIG = 6272
ROW_LAST = N - 15 * ROW_BIG


def _rows_copy(do_copy, rs):
    s = lax.axis_index("s")

    def span(off0, total):
        full, rem = divmod(total, rs)
        for j in range(full):
            do_copy(off0 + j * rs, rs)
        if rem:
            do_copy(off0 + full * rs, rem)

    @pl.when(s < 15)
    def _():
        span(s * ROW_BIG, ROW_BIG)

    @pl.when(s == 15)
    def _():
        span(15 * ROW_BIG, ROW_LAST)


def _edge_pipeline(src_hbm, dst_hbm, h_hbm, acc, bufs, ebase, npairs):
    srcA, dstA, rowsA, srcB, dstB, rowsB, sgA, sgB, ssA, ssB = bufs

    def body(g, carry):
        offA = ebase + (2 * g) * CE
        offB = offA + CE

        @pl.when(g > 0)
        def _():
            pltpu.make_async_copy(h_hbm.at[pl.ds(0, CE), :], rowsA, ssA).wait()
            pltpu.make_async_copy(h_hbm.at[pl.ds(0, CE), :], rowsB, ssB).wait()

        pltpu.sync_copy(src_hbm.at[pl.ds(offA, CE)], srcA)
        pltpu.sync_copy(dst_hbm.at[pl.ds(offA, CE)], dstA)
        gA = pltpu.async_copy(h_hbm.at[srcA], rowsA, sgA)
        pltpu.sync_copy(src_hbm.at[pl.ds(offB, CE)], srcB)
        pltpu.sync_copy(dst_hbm.at[pl.ds(offB, CE)], dstB)
        gB = pltpu.async_copy(h_hbm.at[srcB], rowsB, sgB)
        gA.wait()
        pltpu.async_copy(rowsA, acc.at[dstA], ssA, add=True)
        gB.wait()
        pltpu.async_copy(rowsB, acc.at[dstB], ssB, add=True)
        return carry

    lax.fori_loop(0, npairs, body, 0)
    pltpu.make_async_copy(h_hbm.at[pl.ds(0, CE), :], rowsA, ssA).wait()
    pltpu.make_async_copy(h_hbm.at[pl.ds(0, CE), :], rowsB, ssB).wait()


_AGG_SCRATCH = [
    pltpu.VMEM((CE,), jnp.int32),
    pltpu.VMEM((CE,), jnp.int32),
    pltpu.VMEM((CE, 16), jnp.float32),
    pltpu.VMEM((CE,), jnp.int32),
    pltpu.VMEM((CE,), jnp.int32),
    pltpu.VMEM((CE, 16), jnp.float32),
    pltpu.VMEM_SHARED((N + TRASH, 16), jnp.float32),
    pltpu.SemaphoreType.DMA,
    pltpu.SemaphoreType.DMA,
    pltpu.SemaphoreType.DMA,
    pltpu.SemaphoreType.DMA,
]


@functools.cache
def _sc_kernels():
    mesh = plsc.VectorSubcoreMesh(core_axis_name="c", subcore_axis_name="s")
    cp = pltpu.CompilerParams(use_tc_tiling_on_sc=False)

    deg = functools.partial(
        pl.kernel, mesh=mesh, compiler_params=cp,
        out_type=jax.ShapeDtypeStruct((2 * N,), jnp.float32),
        scratch_types=[
            pltpu.VMEM((C,), jnp.int32),
            pltpu.VMEM((C,), jnp.float32),
            pltpu.VMEM((C,), jnp.float32),
            pltpu.VMEM_SHARED((N,), jnp.float32),
        ],
    )(_deg_kernel)
    agg1 = functools.partial(
        pl.kernel, mesh=mesh, compiler_params=cp,
        out_type=jax.ShapeDtypeStruct((2 * N, 16), jnp.float32),
        scratch_types=list(_AGG_SCRATCH),
    )(_agg1_kernel)
    agg2 = functools.partial(
        pl.kernel, mesh=mesh, compiler_params=cp,
        out_type=tuple(jax.ShapeDtypeStruct((N, 16), jnp.float32)
                       for _ in range(4)),
        scratch_types=list(_AGG_SCRATCH),
    )(_agg2_kernel)
    return deg, agg1, agg2


def _deg_kernel(dst_hbm, zeros_hbm, ones_hbm, out_hbm, dst_v, ones_v, stg, acc):
    c = lax.axis_index("c")
    s = lax.axis_index("s")
    wid = s * 2 + c
    pltpu.sync_copy(ones_hbm, ones_v)
    pltpu.sync_copy(zeros_hbm.at[pl.ds(0, C)], stg)
    _rows_copy(lambda off, n: pltpu.sync_copy(
        stg.at[pl.ds(0, n)], acc.at[pl.ds(off, n)]), C)
    plsc.subcore_barrier()

    def body(i, carry):
        off = wid * (E // 32) + i * C
        pltpu.sync_copy(dst_hbm.at[pl.ds(off, C)], dst_v)
        pltpu.sync_copy(ones_v, acc.at[dst_v], add=True)
        return carry

    lax.fori_loop(0, E // 32 // C, body, 0)
    plsc.subcore_barrier()

    def wb(off, n):
        pltpu.sync_copy(acc.at[pl.ds(off, n)], stg.at[pl.ds(0, n)])
        pltpu.sync_copy(stg.at[pl.ds(0, n)], out_hbm.at[pl.ds(c * N + off, n)])

    _rows_copy(wb, C)


def _agg1_kernel(src_hbm, dst_hbm, xp_hbm, zeros_hbm, out_hbm,
                 srcA, dstA, rowsA, srcB, dstB, rowsB, acc,
                 sgA, sgB, ssA, ssB):
    c = lax.axis_index("c")
    s = lax.axis_index("s")
    wid = s * 2 + c
    pltpu.sync_copy(zeros_hbm.at[pl.ds(0, CE), :], rowsA)
    _rows_copy(lambda off, n: pltpu.sync_copy(
        rowsA.at[pl.ds(0, n), :], acc.at[pl.ds(off, n), :]), CE)
    plsc.subcore_barrier()

    bufs = (srcA, dstA, rowsA, srcB, dstB, rowsB, sgA, sgB, ssA, ssB)
    _edge_pipeline(src_hbm, dst_hbm, xp_hbm, acc, bufs,
                   wid * (E_PAD // 32), E_PAD // 32 // CE // 2)
    plsc.subcore_barrier()

    def wb(off, n):
        pltpu.sync_copy(acc.at[pl.ds(off, n), :], rowsA.at[pl.ds(0, n), :])
        pltpu.sync_copy(rowsA.at[pl.ds(0, n), :],
                        out_hbm.at[pl.ds(c * N + off, n), :])

    _rows_copy(wb, CE)


def _agg2_kernel(src_hbm, dst_hbm, h0_hbm, h1_hbm, h2_hbm, h3_hbm,
                 t0_hbm, t1_hbm, t2_hbm, t3_hbm,
                 srcA, dstA, rowsA, srcB, dstB, rowsB, acc,
                 sgA, sgB, ssA, ssB):
    c = lax.axis_index("c")
    s = lax.axis_index("s")
    h_refs = (h0_hbm, h1_hbm, h2_hbm, h3_hbm)
    t_refs = (t0_hbm, t1_hbm, t2_hbm, t3_hbm)
    for chunk in range(4):
        @pl.when(c == chunk // 2)
        def _(chunk=chunk):
            h_hbm = h_refs[chunk]
            t_hbm = t_refs[chunk]

            def init(off, n):
                pltpu.sync_copy(h_hbm.at[pl.ds(off, n), :],
                                rowsA.at[pl.ds(0, n), :])
                pltpu.sync_copy(rowsA.at[pl.ds(0, n), :],
                                acc.at[pl.ds(off, n), :])

            _rows_copy(init, CE)
            plsc.subcore_barrier()

            bufs = (srcA, dstA, rowsA, srcB, dstB, rowsB, sgA, sgB, ssA, ssB)
            _edge_pipeline(src_hbm, dst_hbm, h_hbm, acc, bufs,
                           s * (E_PAD // 16), E_PAD // 16 // CE // 2)
            plsc.subcore_barrier()

            def wb(off, n):
                pltpu.sync_copy(acc.at[pl.ds(off, n), :],
                                rowsA.at[pl.ds(0, n), :])
                pltpu.sync_copy(rowsA.at[pl.ds(0, n), :],
                                t_hbm.at[pl.ds(off, n), :])

            _rows_copy(wb, CE)


_RB = 2000
_NBLK = N // _RB


def _prep_body(degp_ref, x_ref, dinv_ref, xp_ref):
    deg = degp_ref[0, :, 0] + degp_ref[1, :, 0] + 1.0
    di = lax.rsqrt(deg)
    dinv_ref[...] = di[:, None]
    xp_ref[...] = jnp.concatenate(
        [x_ref[...] * di[:, None],
         jnp.zeros((_RB, 16 - F_IN), jnp.float32)], axis=1)


def _mid_body(sp_ref, xp_ref, dinv_ref, w1_ref, b1_ref, *out_refs):
    di = dinv_ref[...]
    sfull = (sp_ref[0] + sp_ref[1] + xp_ref[...])[:, :F_IN]
    z = jnp.dot(di * sfull, w1_ref[...],
                preferred_element_type=jnp.float32) + b1_ref[...]
    h = jnp.where(z > 0, z, jnp.exp(jnp.minimum(z, 0.0)) - 1.0)
    h1p = di * h
    for k in range(4):
        out_refs[k][...] = h1p[:, 16 * k:16 * (k + 1)]


def _out_body(t0_ref, t1_ref, t2_ref, t3_ref, dinv_ref, batch_ref,
              w2_ref, b2_ref, w3_ref, b3_ref, out_ref, s_acc, c_acc):
    i = pl.program_id(0)
    t = jnp.concatenate(
        [t0_ref[...], t1_ref[...], t2_ref[...], t3_ref[...]], axis=1)
    z = jnp.dot(dinv_ref[...] * t, w2_ref[...],
                preferred_element_type=jnp.float32) + b2_ref[...]
    h2 = jnp.where(z > 0, z, jnp.exp(jnp.minimum(z, 0.0)) - 1.0)
    p = jnp.dot(h2, w3_ref[...], preferred_element_type=jnp.float32)
    onehot = (batch_ref[...] == lax.broadcasted_iota(
        jnp.int32, (_RB, G), 1)).astype(jnp.float32)
    sp = lax.dot_general(onehot, p, (((0,), (0,)), ((), ())),
                         preferred_element_type=jnp.float32)
    cp = jnp.sum(onehot, axis=0)[:, None]

    @pl.when(i == 0)
    def _():
        s_acc[...] = sp
        c_acc[...] = cp

    @pl.when(i > 0)
    def _():
        s_acc[...] += sp
        c_acc[...] += cp

    @pl.when(i == _NBLK - 1)
    def _():
        out_ref[...] = s_acc[...] / jnp.maximum(c_acc[...], 1.0) + b3_ref[...]


def kernel(x, edge_index, batch, W1, b1, W2, b2, W3, b3):
    src = edge_index[0].astype(jnp.int32)
    dst = edge_index[1].astype(jnp.int32)
    batch2 = batch.astype(jnp.int32)[:, None]
    npad = E_PAD - E
    pad_ids = jnp.arange(npad, dtype=jnp.int32)
    src_p = jnp.concatenate([src, (pad_ids * 37) % N])
    dst_p = jnp.concatenate([dst, N + (pad_ids % TRASH)])
    z1 = jnp.zeros((N,), jnp.float32)
    z16 = jnp.zeros((N, 16), jnp.float32)
    ones_c = jnp.ones((C,), jnp.float32)
    b1r = b1[None, :]
    b2r = b2[None, :]
    b3r = b3[None, :]

    deg_k, agg1_k, agg2_k = _sc_kernels()
    degp = deg_k(dst, z1, ones_c)

    dinv, xp = pl.pallas_call(
        _prep_body,
        grid=(_NBLK,),
        in_specs=[
            pl.BlockSpec((2, _RB, 1), lambda i: (0, i, 0)),
            pl.BlockSpec((_RB, F_IN), lambda i: (i, 0)),
        ],
        out_specs=[
            pl.BlockSpec((_RB, 1), lambda i: (i, 0)),
            pl.BlockSpec((_RB, 16), lambda i: (i, 0)),
        ],
        out_shape=[
            jax.ShapeDtypeStruct((N, 1), jnp.float32),
            jax.ShapeDtypeStruct((N, 16), jnp.float32),
        ],
    )(degp.reshape(2, N, 1), x)

    sp = agg1_k(src_p, dst_p, xp, z16)

    h1p = pl.pallas_call(
        _mid_body,
        grid=(_NBLK,),
        in_specs=[
            pl.BlockSpec((2, _RB, 16), lambda i: (0, i, 0)),
            pl.BlockSpec((_RB, 16), lambda i: (i, 0)),
            pl.BlockSpec((_RB, 1), lambda i: (i, 0)),
            pl.BlockSpec((F_IN, H), lambda i: (0, 0)),
            pl.BlockSpec((1, H), lambda i: (0, 0)),
        ],
        out_specs=[pl.BlockSpec((_RB, 16), lambda i: (i, 0))] * 4,
        out_shape=[jax.ShapeDtypeStruct((N, 16), jnp.float32)] * 4,
    )(sp.reshape(2, N, 16), xp, dinv, W1, b1r)

    t = agg2_k(src_p, dst_p, *h1p)

    out = pl.pallas_call(
        _out_body,
        grid=(_NBLK,),
        in_specs=[pl.BlockSpec((_RB, 16), lambda i: (i, 0))] * 4 + [
            pl.BlockSpec((_RB, 1), lambda i: (i, 0)),
            pl.BlockSpec((_RB, 1), lambda i: (i, 0)),
            pl.BlockSpec((H, H), lambda i: (0, 0)),
            pl.BlockSpec((1, H), lambda i: (0, 0)),
            pl.BlockSpec((H, 1), lambda i: (0, 0)),
            pl.BlockSpec((1, 1), lambda i: (0, 0)),
        ],
        out_specs=pl.BlockSpec((G, 1), lambda i: (0, 0)),
        out_shape=jax.ShapeDtypeStruct((G, 1), jnp.float32),
        scratch_shapes=[
            pltpu.VMEM((G, 1), jnp.float32),
            pltpu.VMEM((G, 1), jnp.float32),
        ],
    )(*t, dinv, batch2, W2, b2r, W3, b3r)

    return out


# speedup vs baseline: 26.6945x; 1.0323x over previous
import functools

import jax
import jax.numpy as jnp
from jax import lax
from jax.experimental import pallas as pl
from jax.experimental.pallas import tpu as pltpu
from jax.experimental.pallas import tpu_sc as plsc

N = 100000
E = 1600000
F_IN = 4
H = 64
G = 1024

CE = 800
E_PAD = 1638400
TRASH = 1024
C = 2000
NP8 = 100352
ROW_BIG = 6272
ROW_LAST = N - 15 * ROW_BIG


def _rows_copy(do_copy, rs):
    s = lax.axis_index("s")

    def span(off0, total):
        full, rem = divmod(total, rs)
        for j in range(full):
            do_copy(off0 + j * rs, rs)
        if rem:
            do_copy(off0 + full * rs, rem)

    @pl.when(s < 15)
    def _():
        span(s * ROW_BIG, ROW_BIG)

    @pl.when(s == 15)
    def _():
        span(15 * ROW_BIG, ROW_LAST)


def _edge_pipeline(eidx_hbm, h_hbm, acc, bufs, cbase, npairs):
    idxA, rowsA, idxB, rowsB, sgA, sgB, ssA, ssB = bufs

    def body(g, carry):
        kA = cbase + 2 * g
        kB = kA + 1

        @pl.when(g > 0)
        def _():
            pltpu.make_async_copy(h_hbm.at[pl.ds(0, CE), :], rowsA, ssA).wait()
            pltpu.make_async_copy(h_hbm.at[pl.ds(0, CE), :], rowsB, ssB).wait()

        pltpu.sync_copy(eidx_hbm.at[kA], idxA)
        gA = pltpu.async_copy(h_hbm.at[idxA.at[0]], rowsA, sgA)
        pltpu.sync_copy(eidx_hbm.at[kB], idxB)
        gB = pltpu.async_copy(h_hbm.at[idxB.at[0]], rowsB, sgB)
        gA.wait()
        pltpu.async_copy(rowsA, acc.at[idxA.at[1]], ssA, add=True)
        gB.wait()
        pltpu.async_copy(rowsB, acc.at[idxB.at[1]], ssB, add=True)
        return carry

    lax.fori_loop(0, npairs, body, 0)
    pltpu.make_async_copy(h_hbm.at[pl.ds(0, CE), :], rowsA, ssA).wait()
    pltpu.make_async_copy(h_hbm.at[pl.ds(0, CE), :], rowsB, ssB).wait()


_AGG_SCRATCH = [
    pltpu.VMEM((2, CE), jnp.int32),
    pltpu.VMEM((CE, 16), jnp.float32),
    pltpu.VMEM((2, CE), jnp.int32),
    pltpu.VMEM((CE, 16), jnp.float32),
    pltpu.VMEM_SHARED((N + TRASH, 16), jnp.float32),
    pltpu.SemaphoreType.DMA,
    pltpu.SemaphoreType.DMA,
    pltpu.SemaphoreType.DMA,
    pltpu.SemaphoreType.DMA,
]


@functools.cache
def _sc_kernels():
    mesh = plsc.VectorSubcoreMesh(core_axis_name="c", subcore_axis_name="s")
    cp = pltpu.CompilerParams(use_tc_tiling_on_sc=False)

    deg = functools.partial(
        pl.kernel, mesh=mesh, compiler_params=cp,
        out_type=jax.ShapeDtypeStruct((2 * N,), jnp.float32),
        scratch_types=[
            pltpu.VMEM((C,), jnp.int32),
            pltpu.VMEM((C,), jnp.float32),
            pltpu.VMEM((C,), jnp.float32),
            pltpu.VMEM_SHARED((N,), jnp.float32),
        ],
    )(_deg_kernel)
    agg1 = functools.partial(
        pl.kernel, mesh=mesh, compiler_params=cp,
        out_type=jax.ShapeDtypeStruct((2 * N, 16), jnp.float32),
        scratch_types=list(_AGG_SCRATCH),
    )(_agg1_kernel)
    agg2 = functools.partial(
        pl.kernel, mesh=mesh, compiler_params=cp,
        out_type=tuple(jax.ShapeDtypeStruct((N, 16), jnp.float32)
                       for _ in range(4)),
        scratch_types=list(_AGG_SCRATCH),
    )(_agg2_kernel)
    return deg, agg1, agg2


def _deg_kernel(dst_hbm, zeros_hbm, ones_hbm, out_hbm, dst_v, ones_v, stg, acc):
    c = lax.axis_index("c")
    s = lax.axis_index("s")
    wid = s * 2 + c
    pltpu.sync_copy(ones_hbm, ones_v)
    pltpu.sync_copy(zeros_hbm.at[pl.ds(0, C)], stg)
    _rows_copy(lambda off, n: pltpu.sync_copy(
        stg.at[pl.ds(0, n)], acc.at[pl.ds(off, n)]), C)
    plsc.subcore_barrier()

    def body(i, carry):
        off = wid * (E // 32) + i * C
        pltpu.sync_copy(dst_hbm.at[pl.ds(off, C)], dst_v)
        pltpu.sync_copy(ones_v, acc.at[dst_v], add=True)
        return carry

    lax.fori_loop(0, E // 32 // C, body, 0)
    plsc.subcore_barrier()

    def wb(off, n):
        pltpu.sync_copy(acc.at[pl.ds(off, n)], stg.at[pl.ds(0, n)])
        pltpu.sync_copy(stg.at[pl.ds(0, n)], out_hbm.at[pl.ds(c * N + off, n)])

    _rows_copy(wb, C)


def _agg1_kernel(eidx_hbm, xp_hbm, zeros_hbm, out_hbm,
                 idxA, rowsA, idxB, rowsB, acc,
                 sgA, sgB, ssA, ssB):
    c = lax.axis_index("c")
    s = lax.axis_index("s")
    wid = s * 2 + c
    pltpu.sync_copy(zeros_hbm.at[pl.ds(0, CE), :], rowsA)
    _rows_copy(lambda off, n: pltpu.sync_copy(
        rowsA.at[pl.ds(0, n), :], acc.at[pl.ds(off, n), :]), CE)
    plsc.subcore_barrier()

    bufs = (idxA, rowsA, idxB, rowsB, sgA, sgB, ssA, ssB)
    npairs = E_PAD // 32 // CE // 2
    _edge_pipeline(eidx_hbm, xp_hbm, acc, bufs, wid * 2 * npairs, npairs)
    plsc.subcore_barrier()

    def wb(off, n):
        pltpu.sync_copy(acc.at[pl.ds(off, n), :], rowsA.at[pl.ds(0, n), :])
        pltpu.sync_copy(rowsA.at[pl.ds(0, n), :],
                        out_hbm.at[pl.ds(c * N + off, n), :])

    _rows_copy(wb, CE)


def _agg2_kernel(eidx_hbm, h0_hbm, h1_hbm, h2_hbm, h3_hbm,
                 t0_hbm, t1_hbm, t2_hbm, t3_hbm,
                 idxA, rowsA, idxB, rowsB, acc,
                 sgA, sgB, ssA, ssB):
    c = lax.axis_index("c")
    s = lax.axis_index("s")
    h_refs = (h0_hbm, h1_hbm, h2_hbm, h3_hbm)
    t_refs = (t0_hbm, t1_hbm, t2_hbm, t3_hbm)
    for chunk in range(4):
        @pl.when(c == chunk // 2)
        def _(chunk=chunk):
            h_hbm = h_refs[chunk]
            t_hbm = t_refs[chunk]

            def init(off, n):
                pltpu.sync_copy(h_hbm.at[pl.ds(off, n), :],
                                rowsA.at[pl.ds(0, n), :])
                pltpu.sync_copy(rowsA.at[pl.ds(0, n), :],
                                acc.at[pl.ds(off, n), :])

            _rows_copy(init, CE)
            plsc.subcore_barrier()

            bufs = (idxA, rowsA, idxB, rowsB, sgA, sgB, ssA, ssB)
            npairs = E_PAD // 16 // CE // 2
            _edge_pipeline(eidx_hbm, h_hbm, acc, bufs, s * 2 * npairs, npairs)
            plsc.subcore_barrier()

            def wb(off, n):
                pltpu.sync_copy(acc.at[pl.ds(off, n), :],
                                rowsA.at[pl.ds(0, n), :])
                pltpu.sync_copy(rowsA.at[pl.ds(0, n), :],
                                t_hbm.at[pl.ds(off, n), :])

            _rows_copy(wb, CE)


_RB = 2000
_NBLK = N // _RB


def _prep_body(degp_ref, x_ref, dinv_ref, xp_ref):
    deg = degp_ref[0, :, 0] + degp_ref[1, :, 0] + 1.0
    di = lax.rsqrt(deg)
    dinv_ref[...] = di[:, None]
    xp_ref[...] = jnp.concatenate(
        [x_ref[...] * di[:, None],
         jnp.zeros((_RB, 16 - F_IN), jnp.float32)], axis=1)


def _mid_body(sp_ref, xp_ref, dinv_ref, w1_ref, b1_ref, *out_refs):
    di = dinv_ref[...]
    sfull = (sp_ref[0] + sp_ref[1] + xp_ref[...])[:, :F_IN]
    z = jnp.dot(di * sfull, w1_ref[...],
                preferred_element_type=jnp.float32) + b1_ref[...]
    h = jnp.where(z > 0, z, jnp.exp(jnp.minimum(z, 0.0)) - 1.0)
    h1p = di * h
    for k in range(4):
        out_refs[k][...] = h1p[:, 16 * k:16 * (k + 1)]


def _out_body(t0_ref, t1_ref, t2_ref, t3_ref, dinv_ref, batch_ref,
              w2_ref, b2_ref, w3_ref, b3_ref, out_ref, s_acc):
    i = pl.program_id(0)
    t = jnp.concatenate(
        [t0_ref[...], t1_ref[...], t2_ref[...], t3_ref[...]], axis=1)
    z = jnp.dot(dinv_ref[...] * t, w2_ref[...],
                preferred_element_type=jnp.float32) + b2_ref[...]
    h2 = jnp.where(z > 0, z, jnp.exp(jnp.minimum(z, 0.0)) - 1.0)
    p = jnp.dot(h2, w3_ref[...], preferred_element_type=jnp.float32)
    onehot = (batch_ref[...] == lax.broadcasted_iota(
        jnp.int32, (_RB, G), 1)).astype(jnp.float32)
    pc = jnp.concatenate([p, jnp.ones_like(p)], axis=1)
    sp = lax.dot_general(onehot, pc, (((0,), (0,)), ((), ())),
                         preferred_element_type=jnp.float32)

    @pl.when(i == 0)
    def _():
        s_acc[...] = sp

    @pl.when(i > 0)
    def _():
        s_acc[...] += sp

    @pl.when(i == _NBLK - 1)
    def _():
        out_ref[...] = (s_acc[:, 0:1] / jnp.maximum(s_acc[:, 1:2], 1.0)
                        + b3_ref[...])


def kernel(x, edge_index, batch, W1, b1, W2, b2, W3, b3):
    src = edge_index[0].astype(jnp.int32)
    dst = edge_index[1].astype(jnp.int32)
    batch2 = batch.astype(jnp.int32)[:, None]
    npad = E_PAD - E
    pad_ids = jnp.arange(npad, dtype=jnp.int32)
    src_p = jnp.concatenate([src, (pad_ids * 37) % N])
    dst_p = jnp.concatenate([dst, N + (pad_ids % TRASH)])
    eidx = jnp.stack([src_p.reshape(-1, CE), dst_p.reshape(-1, CE)], axis=1)
    z1 = jnp.zeros((N,), jnp.float32)
    z16 = jnp.zeros((N, 16), jnp.float32)
    ones_c = jnp.ones((C,), jnp.float32)
    b1r = b1[None, :]
    b2r = b2[None, :]
    b3r = b3[None, :]

    deg_k, agg1_k, agg2_k = _sc_kernels()
    degp = deg_k(dst, z1, ones_c)

    dinv, xp = pl.pallas_call(
        _prep_body,
        grid=(_NBLK,),
        in_specs=[
            pl.BlockSpec((2, _RB, 1), lambda i: (0, i, 0)),
            pl.BlockSpec((_RB, F_IN), lambda i: (i, 0)),
        ],
        out_specs=[
            pl.BlockSpec((_RB, 1), lambda i: (i, 0)),
            pl.BlockSpec((_RB, 16), lambda i: (i, 0)),
        ],
        out_shape=[
            jax.ShapeDtypeStruct((N, 1), jnp.float32),
            jax.ShapeDtypeStruct((N, 16), jnp.float32),
        ],
    )(degp.reshape(2, N, 1), x)

    sp = agg1_k(eidx, xp, z16)

    h1p = pl.pallas_call(
        _mid_body,
        grid=(_NBLK,),
        in_specs=[
            pl.BlockSpec((2, _RB, 16), lambda i: (0, i, 0)),
            pl.BlockSpec((_RB, 16), lambda i: (i, 0)),
            pl.BlockSpec((_RB, 1), lambda i: (i, 0)),
            pl.BlockSpec((F_IN, H), lambda i: (0, 0)),
            pl.BlockSpec((1, H), lambda i: (0, 0)),
        ],
        out_specs=[pl.BlockSpec((_RB, 16), lambda i: (i, 0))] * 4,
        out_shape=[jax.ShapeDtypeStruct((N, 16), jnp.float32)] * 4,
    )(sp.reshape(2, N, 16), xp, dinv, W1, b1r)

    t = agg2_k(eidx, *h1p)

    out = pl.pallas_call(
        _out_body,
        grid=(_NBLK,),
        in_specs=[pl.BlockSpec((_RB, 16), lambda i: (i, 0))] * 4 + [
            pl.BlockSpec((_RB, 1), lambda i: (i, 0)),
            pl.BlockSpec((_RB, 1), lambda i: (i, 0)),
            pl.BlockSpec((H, H), lambda i: (0, 0)),
            pl.BlockSpec((1, H), lambda i: (0, 0)),
            pl.BlockSpec((H, 1), lambda i: (0, 0)),
            pl.BlockSpec((1, 1), lambda i: (0, 0)),
        ],
        out_specs=pl.BlockSpec((G, 1), lambda i: (0, 0)),
        out_shape=jax.ShapeDtypeStruct((G, 1), jnp.float32),
        scratch_shapes=[
            pltpu.VMEM((G, 2), jnp.float32),
        ],
    )(*t, dinv, batch2, W2, b2r, W3, b3r)

    return out


# speedup vs baseline: 26.7895x; 1.0036x over previous
import functools

import jax
import jax.numpy as jnp
from jax import lax
from jax.experimental import pallas as pl
from jax.experimental.pallas import tpu as pltpu
from jax.experimental.pallas import tpu_sc as plsc

N = 100000
E = 1600000
F_IN = 4
H = 64
G = 1024

CE = 800
E_PAD = 1638400
TRASH = 1024
C = 2000
NP8 = 100352
ROW_BIG = 6272
ROW_LAST = N - 15 * ROW_BIG


def _rows_copy(do_copy, rs):
    s = lax.axis_index("s")

    def span(off0, total):
        full, rem = divmod(total, rs)
        for j in range(full):
            do_copy(off0 + j * rs, rs)
        if rem:
            do_copy(off0 + full * rs, rem)

    @pl.when(s < 15)
    def _():
        span(s * ROW_BIG, ROW_BIG)

    @pl.when(s == 15)
    def _():
        span(15 * ROW_BIG, ROW_LAST)


def _edge_pipeline(eidx_hbm, h_hbm, acc, bufs, cbase, npairs):
    idxA, rowsA, idxB, rowsB, sgA, sgB, ssA, ssB = bufs

    def body(g, carry):
        kA = cbase + 2 * g
        kB = kA + 1

        @pl.when(g > 0)
        def _():
            pltpu.make_async_copy(h_hbm.at[pl.ds(0, CE), :], rowsA, ssA).wait()
            pltpu.make_async_copy(h_hbm.at[pl.ds(0, CE), :], rowsB, ssB).wait()

        pltpu.sync_copy(eidx_hbm.at[kA], idxA)
        gA = pltpu.async_copy(h_hbm.at[idxA.at[0]], rowsA, sgA)
        pltpu.sync_copy(eidx_hbm.at[kB], idxB)
        gB = pltpu.async_copy(h_hbm.at[idxB.at[0]], rowsB, sgB)
        gA.wait()
        pltpu.async_copy(rowsA, acc.at[idxA.at[1]], ssA, add=True)
        gB.wait()
        pltpu.async_copy(rowsB, acc.at[idxB.at[1]], ssB, add=True)
        return carry

    lax.fori_loop(0, npairs, body, 0)
    pltpu.make_async_copy(h_hbm.at[pl.ds(0, CE), :], rowsA, ssA).wait()
    pltpu.make_async_copy(h_hbm.at[pl.ds(0, CE), :], rowsB, ssB).wait()


_AGG_SCRATCH = [
    pltpu.VMEM((2, CE), jnp.int32),
    pltpu.VMEM((CE, 16), jnp.float32),
    pltpu.VMEM((2, CE), jnp.int32),
    pltpu.VMEM((CE, 16), jnp.float32),
    pltpu.VMEM_SHARED((N + TRASH, 16), jnp.float32),
    pltpu.SemaphoreType.DMA,
    pltpu.SemaphoreType.DMA,
    pltpu.SemaphoreType.DMA,
    pltpu.SemaphoreType.DMA,
]


@functools.cache
def _sc_kernels():
    mesh = plsc.VectorSubcoreMesh(core_axis_name="c", subcore_axis_name="s")
    cp = pltpu.CompilerParams(use_tc_tiling_on_sc=False)

    deg = functools.partial(
        pl.kernel, mesh=mesh, compiler_params=cp,
        out_type=jax.ShapeDtypeStruct((2 * N,), jnp.float32),
        scratch_types=[
            pltpu.VMEM((C,), jnp.int32),
            pltpu.VMEM((C,), jnp.float32),
            pltpu.VMEM((C,), jnp.float32),
            pltpu.VMEM_SHARED((N,), jnp.float32),
        ],
    )(_deg_kernel)
    agg1 = functools.partial(
        pl.kernel, mesh=mesh, compiler_params=cp,
        out_type=jax.ShapeDtypeStruct((2 * N, 16), jnp.float32),
        scratch_types=list(_AGG_SCRATCH),
    )(_agg1_kernel)
    agg2 = functools.partial(
        pl.kernel, mesh=mesh, compiler_params=cp,
        out_type=tuple(jax.ShapeDtypeStruct((N, 16), jnp.float32)
                       for _ in range(4)),
        scratch_types=list(_AGG_SCRATCH),
    )(_agg2_kernel)
    return deg, agg1, agg2


def _deg_kernel(dst_hbm, zeros_hbm, ones_hbm, out_hbm, dst_v, ones_v, stg, acc):
    c = lax.axis_index("c")
    s = lax.axis_index("s")
    wid = s * 2 + c
    pltpu.sync_copy(ones_hbm, ones_v)
    pltpu.sync_copy(zeros_hbm.at[pl.ds(0, C)], stg)
    _rows_copy(lambda off, n: pltpu.sync_copy(
        stg.at[pl.ds(0, n)], acc.at[pl.ds(off, n)]), C)
    plsc.subcore_barrier()

    def body(i, carry):
        off = wid * (E // 32) + i * C
        pltpu.sync_copy(dst_hbm.at[pl.ds(off, C)], dst_v)
        pltpu.sync_copy(ones_v, acc.at[dst_v], add=True)
        return carry

    lax.fori_loop(0, E // 32 // C, body, 0)
    plsc.subcore_barrier()

    def wb(off, n):
        pltpu.sync_copy(acc.at[pl.ds(off, n)], stg.at[pl.ds(0, n)])
        pltpu.sync_copy(stg.at[pl.ds(0, n)], out_hbm.at[pl.ds(c * N + off, n)])

    _rows_copy(wb, C)


def _agg1_kernel(eidx_hbm, xp_hbm, zeros_hbm, out_hbm,
                 idxA, rowsA, idxB, rowsB, acc,
                 sgA, sgB, ssA, ssB):
    c = lax.axis_index("c")
    s = lax.axis_index("s")
    wid = s * 2 + c
    pltpu.sync_copy(zeros_hbm.at[pl.ds(0, CE), :], rowsA)
    _rows_copy(lambda off, n: pltpu.sync_copy(
        rowsA.at[pl.ds(0, n), :], acc.at[pl.ds(off, n), :]), CE)
    plsc.subcore_barrier()

    bufs = (idxA, rowsA, idxB, rowsB, sgA, sgB, ssA, ssB)
    npairs = E_PAD // 32 // CE // 2
    _edge_pipeline(eidx_hbm, xp_hbm, acc, bufs, wid * 2 * npairs, npairs)
    plsc.subcore_barrier()

    def wb(off, n):
        pltpu.sync_copy(acc.at[pl.ds(off, n), :], rowsA.at[pl.ds(0, n), :])
        pltpu.sync_copy(rowsA.at[pl.ds(0, n), :],
                        out_hbm.at[pl.ds(c * N + off, n), :])

    _rows_copy(wb, CE)


def _agg2_kernel(eidx_hbm, h0_hbm, h1_hbm, h2_hbm, h3_hbm,
                 t0_hbm, t1_hbm, t2_hbm, t3_hbm,
                 idxA, rowsA, idxB, rowsB, acc,
                 sgA, sgB, ssA, ssB):
    c = lax.axis_index("c")
    s = lax.axis_index("s")
    h_refs = (h0_hbm, h1_hbm, h2_hbm, h3_hbm)
    t_refs = (t0_hbm, t1_hbm, t2_hbm, t3_hbm)
    for chunk in range(4):
        @pl.when(c == chunk // 2)
        def _(chunk=chunk):
            h_hbm = h_refs[chunk]
            t_hbm = t_refs[chunk]

            def init(off, n):
                pltpu.sync_copy(h_hbm.at[pl.ds(off, n), :],
                                rowsA.at[pl.ds(0, n), :])
                pltpu.sync_copy(rowsA.at[pl.ds(0, n), :],
                                acc.at[pl.ds(off, n), :])

            _rows_copy(init, CE)
            plsc.subcore_barrier()

            bufs = (idxA, rowsA, idxB, rowsB, sgA, sgB, ssA, ssB)
            npairs = E_PAD // 16 // CE // 2
            _edge_pipeline(eidx_hbm, h_hbm, acc, bufs, s * 2 * npairs, npairs)
            plsc.subcore_barrier()

            def wb(off, n):
                pltpu.sync_copy(acc.at[pl.ds(off, n), :],
                                rowsA.at[pl.ds(0, n), :])
                pltpu.sync_copy(rowsA.at[pl.ds(0, n), :],
                                t_hbm.at[pl.ds(off, n), :])

            _rows_copy(wb, CE)


_RB = 4000
_NBLK = N // _RB


def _prep_body(degp_ref, x_ref, dinv_ref, xp_ref):
    deg = degp_ref[0, :, 0] + degp_ref[1, :, 0] + 1.0
    di = lax.rsqrt(deg)
    dinv_ref[...] = di[:, None]
    xp_ref[...] = jnp.concatenate(
        [x_ref[...] * di[:, None],
         jnp.zeros((_RB, 16 - F_IN), jnp.float32)], axis=1)


def _mid_body(sp_ref, xp_ref, dinv_ref, w1_ref, b1_ref, *out_refs):
    di = dinv_ref[...]
    sfull = (sp_ref[0] + sp_ref[1] + xp_ref[...])[:, :F_IN]
    z = jnp.dot(di * sfull, w1_ref[...],
                preferred_element_type=jnp.float32) + b1_ref[...]
    h = jnp.where(z > 0, z, jnp.exp(jnp.minimum(z, 0.0)) - 1.0)
    h1p = di * h
    for k in range(4):
        out_refs[k][...] = h1p[:, 16 * k:16 * (k + 1)]


def _out_body(t0_ref, t1_ref, t2_ref, t3_ref, dinv_ref, batch_ref,
              w2_ref, b2_ref, w3_ref, b3_ref, out_ref, s_acc):
    i = pl.program_id(0)
    t = jnp.concatenate(
        [t0_ref[...], t1_ref[...], t2_ref[...], t3_ref[...]], axis=1)
    z = jnp.dot(dinv_ref[...] * t, w2_ref[...],
                preferred_element_type=jnp.float32) + b2_ref[...]
    h2 = jnp.where(z > 0, z, jnp.exp(jnp.minimum(z, 0.0)) - 1.0)
    p = jnp.dot(h2, w3_ref[...], preferred_element_type=jnp.float32)
    onehot = (batch_ref[...] == lax.broadcasted_iota(
        jnp.int32, (_RB, G), 1)).astype(jnp.float32)
    pc = jnp.concatenate([p, jnp.ones_like(p)], axis=1)
    sp = lax.dot_general(onehot, pc, (((0,), (0,)), ((), ())),
                         preferred_element_type=jnp.float32)

    @pl.when(i == 0)
    def _():
        s_acc[...] = sp

    @pl.when(i > 0)
    def _():
        s_acc[...] += sp

    @pl.when(i == _NBLK - 1)
    def _():
        out_ref[...] = (s_acc[:, 0:1] / jnp.maximum(s_acc[:, 1:2], 1.0)
                        + b3_ref[...])


def kernel(x, edge_index, batch, W1, b1, W2, b2, W3, b3):
    src = edge_index[0].astype(jnp.int32)
    dst = edge_index[1].astype(jnp.int32)
    batch2 = batch.astype(jnp.int32)[:, None]
    npad = E_PAD - E
    pad_ids = jnp.arange(npad, dtype=jnp.int32)
    src_p = jnp.concatenate([src, (pad_ids * 37) % N])
    dst_p = jnp.concatenate([dst, N + (pad_ids % TRASH)])
    eidx = jnp.stack([src_p.reshape(-1, CE), dst_p.reshape(-1, CE)], axis=1)
    z1 = jnp.zeros((N,), jnp.float32)
    z16 = jnp.zeros((N, 16), jnp.float32)
    ones_c = jnp.ones((C,), jnp.float32)
    b1r = b1[None, :]
    b2r = b2[None, :]
    b3r = b3[None, :]

    deg_k, agg1_k, agg2_k = _sc_kernels()
    degp = deg_k(dst, z1, ones_c)

    dinv, xp = pl.pallas_call(
        _prep_body,
        grid=(_NBLK,),
        in_specs=[
            pl.BlockSpec((2, _RB, 1), lambda i: (0, i, 0)),
            pl.BlockSpec((_RB, F_IN), lambda i: (i, 0)),
        ],
        out_specs=[
            pl.BlockSpec((_RB, 1), lambda i: (i, 0)),
            pl.BlockSpec((_RB, 16), lambda i: (i, 0)),
        ],
        out_shape=[
            jax.ShapeDtypeStruct((N, 1), jnp.float32),
            jax.ShapeDtypeStruct((N, 16), jnp.float32),
        ],
    )(degp.reshape(2, N, 1), x)

    sp = agg1_k(eidx, xp, z16)

    h1p = pl.pallas_call(
        _mid_body,
        grid=(_NBLK,),
        in_specs=[
            pl.BlockSpec((2, _RB, 16), lambda i: (0, i, 0)),
            pl.BlockSpec((_RB, 16), lambda i: (i, 0)),
            pl.BlockSpec((_RB, 1), lambda i: (i, 0)),
            pl.BlockSpec((F_IN, H), lambda i: (0, 0)),
            pl.BlockSpec((1, H), lambda i: (0, 0)),
        ],
        out_specs=[pl.BlockSpec((_RB, 16), lambda i: (i, 0))] * 4,
        out_shape=[jax.ShapeDtypeStruct((N, 16), jnp.float32)] * 4,
    )(sp.reshape(2, N, 16), xp, dinv, W1, b1r)

    t = agg2_k(eidx, *h1p)

    out = pl.pallas_call(
        _out_body,
        grid=(_NBLK,),
        in_specs=[pl.BlockSpec((_RB, 16), lambda i: (i, 0))] * 4 + [
            pl.BlockSpec((_RB, 1), lambda i: (i, 0)),
            pl.BlockSpec((_RB, 1), lambda i: (i, 0)),
            pl.BlockSpec((H, H), lambda i: (0, 0)),
            pl.BlockSpec((1, H), lambda i: (0, 0)),
            pl.BlockSpec((H, 1), lambda i: (0, 0)),
            pl.BlockSpec((1, 1), lambda i: (0, 0)),
        ],
        out_specs=pl.BlockSpec((G, 1), lambda i: (0, 0)),
        out_shape=jax.ShapeDtypeStruct((G, 1), jnp.float32),
        scratch_shapes=[
            pltpu.VMEM((G, 2), jnp.float32),
        ],
    )(*t, dinv, batch2, W2, b2r, W3, b3r)

    return out
